# Initial kernel scaffold; baseline (speedup 1.0000x reference)
import jax
import jax.numpy as jnp
from jax.experimental import pallas as pl


def kernel(x):
    raise NotImplementedError("write your pallas kernel here")


# speedup vs baseline: 3.2147x; 3.2147x over previous
import jax
import jax.numpy as jnp
from jax.experimental import pallas as pl

_K = 64
_N = 32768
_ROWS = 128
_RB = 8

def _topk_mask_body(x_ref, o_ref):
    _MIN32 = jnp.int32(-2147483648)
    x = x_ref[...]
    k = jax.lax.bitcast_convert_type(x, jnp.int32)
    s = jnp.where(k >= 0, k, k ^ jnp.int32(0x7FFFFFFF))

    def bit_step(i, p_u):
        b = 31 - i
        cand_u = p_u | (jnp.int32(1) << b)
        cand_s = cand_u ^ _MIN32
        cnt = jnp.sum((s >= cand_s).astype(jnp.int32), axis=1, keepdims=True)
        return jnp.where(cnt >= _K, cand_u, p_u)

    p_u = jax.lax.fori_loop(0, 32, bit_step,
                            jnp.zeros((x.shape[0], 1), jnp.int32))
    v_s = p_u ^ _MIN32

    gt = s > v_s
    eq = s == v_s
    c_gt = jnp.sum(gt.astype(jnp.int32), axis=1, keepdims=True)
    m = _K - c_gt

    col = jax.lax.broadcasted_iota(jnp.int32, x.shape, 1)
    eqi = eq.astype(jnp.int32)

    def idx_step(i, carry):
        lo, hi = carry
        mid = (lo + hi) >> 1
        cnt = jnp.sum(jnp.where(col <= mid, eqi, 0), axis=1, keepdims=True)
        take = cnt >= m
        return jnp.where(take, lo, mid + 1), jnp.where(take, mid, hi)

    lo0 = jnp.zeros((x.shape[0], 1), jnp.int32)
    hi0 = jnp.full((x.shape[0], 1), _N - 1, jnp.int32)
    lo, hi = jax.lax.fori_loop(0, 15, idx_step, (lo0, hi0))

    mask = gt | (eq & (col <= hi))
    o_ref[...] = jnp.where(mask, jnp.maximum(x, 0.0), 0.0)


def kernel(x):
    return pl.pallas_call(
        _topk_mask_body,
        grid=(_ROWS // _RB,),
        in_specs=[pl.BlockSpec((_RB, _N), lambda i: (i, 0))],
        out_specs=pl.BlockSpec((_RB, _N), lambda i: (i, 0)),
        out_shape=jax.ShapeDtypeStruct((_ROWS, _N), jnp.float32),
    )(x)


# speedup vs baseline: 4.0629x; 1.2639x over previous
import functools

import jax
import jax.numpy as jnp
from jax import lax
from jax.experimental import pallas as pl
from jax.experimental.pallas import tpu as pltpu
from jax.experimental.pallas import tpu_sc as plsc

_K = 64
_N = 32768
_ROWS = 128
_NC = 2
_NS = 16
_NW = _NC * _NS
_RPW = _ROWS // _NW
_CHUNKS = _N // 16
_CKPTS = 32
_CPC = _CHUNKS // _CKPTS
_CAP = _CKPTS * 16 + _CPC * 16 + 16
_TRIG = 256
_IMIN = -2147483648


def _sortable(v):
    k = lax.bitcast_convert_type(v, jnp.int32)
    return jnp.where(k >= 0, k, k ^ jnp.int32(0x7FFFFFFF))


def _unsortable(s):
    k = jnp.where(s >= 0, s, s ^ jnp.int32(0x7FFFFFFF))
    return lax.bitcast_convert_type(k, jnp.float32)


def _splat(x, dtype=jnp.int32):
    return lax.broadcast(jnp.asarray(x, dtype), (16,))


def _rebuild(buf, candi, sbuf, ctmp, n_splat):
    iota16 = lax.iota(jnp.int32, 16)
    zeros_i = jnp.zeros((16,), jnp.int32)
    n_s = jnp.max(n_splat)
    nv = (n_s + 15) // 16

    def g_body(vi, _):
        base = vi * 16
        idxv = candi[pl.ds(base, 16)]
        valid = (lax.broadcast(base, (16,)) + iota16) < n_splat
        vals = plsc.load_gather(buf, [idxv], mask=valid)
        s = jnp.where(valid, _sortable(vals), jnp.int32(_IMIN))
        sbuf[pl.ds(base, 16)] = s
        return 0

    lax.fori_loop(0, nv, g_body, 0)

    def bit_body(i, p_u):
        cand_u = p_u | _splat(jnp.int32(1) << (31 - i))
        cand_s = cand_u ^ jnp.int32(_IMIN)

        def cnt_body(vi, acc):
            s = sbuf[pl.ds(vi * 16, 16)]
            return acc + plsc.all_reduce_population_count(s >= cand_s)

        cnt = lax.fori_loop(0, nv, cnt_body, zeros_i)
        return jnp.where(cnt >= _K, cand_u, p_u)

    p_u = lax.fori_loop(0, 32, bit_body, zeros_i)
    v64s = p_u ^ jnp.int32(_IMIN)

    def cgt_body(vi, acc):
        s = sbuf[pl.ds(vi * 16, 16)]
        return acc + plsc.all_reduce_population_count(s > v64s)

    c_gt = lax.fori_loop(0, nv, cgt_body, zeros_i)
    m_allow = _splat(_K) - c_gt

    def cpt_body(vi, carry):
        eq_seen, nn = carry
        base = vi * 16
        s = sbuf[pl.ds(base, 16)]
        idxv = candi[pl.ds(base, 16)]
        gt = s > v64s
        eq = s == v64s
        cum_eq = jnp.cumsum(jnp.where(eq, 1, 0))
        keep = gt | (eq & ((cum_eq + eq_seen) <= m_allow))
        slot = nn + jnp.cumsum(jnp.where(keep, 1, 0)) - 1
        plsc.store_scatter(ctmp, [slot], idxv, mask=keep)
        return (eq_seen + plsc.all_reduce_population_count(eq),
                nn + plsc.all_reduce_population_count(keep))

    lax.fori_loop(0, nv, cpt_body, (zeros_i, zeros_i))

    for i in range(_K // 16):
        candi[pl.ds(i * 16, 16)] = ctmp[pl.ds(i * 16, 16)]
    return _splat(_K), _unsortable(v64s)


def _scan_row(buf, candi, sbuf, ctmp):
    iota16 = lax.iota(jnp.int32, 16)

    def ckpt_body(kc, carry):
        n_splat, thresh = carry

        def chunk_body(c, n_sp):
            v = buf[pl.ds(c * 16, 16)]
            m = v > thresh
            slot = n_sp + jnp.cumsum(jnp.where(m, 1, 0)) - 1
            idxv = iota16 + lax.broadcast(c * 16, (16,))
            plsc.store_scatter(candi, [slot], idxv, mask=m)
            return n_sp + plsc.all_reduce_population_count(m)

        n_splat = lax.fori_loop(kc * _CPC, (kc + 1) * _CPC, chunk_body,
                                n_splat)
        n_s = jnp.max(n_splat)
        return lax.cond(
            n_s >= _TRIG,
            lambda c: _rebuild(buf, candi, sbuf, ctmp, c[0]),
            lambda c: c,
            (n_splat, thresh))

    n0 = jnp.zeros((16,), jnp.int32)
    t0 = jnp.full((16,), -jnp.inf, jnp.float32)
    n_splat, _ = lax.fori_loop(0, _CKPTS, ckpt_body, (n0, t0))
    _rebuild(buf, candi, sbuf, ctmp, n_splat)


_mesh = plsc.VectorSubcoreMesh(core_axis_name="c", subcore_axis_name="s")


_KERNEL_KWARGS = dict(
    mesh=_mesh,
    compiler_params=pltpu.CompilerParams(needs_layout_passes=False),
    out_type=jax.ShapeDtypeStruct((_ROWS, _N), jnp.float32),
    scratch_types=[
        pltpu.VMEM((_N,), jnp.float32),
        pltpu.VMEM((_N,), jnp.float32),
        pltpu.VMEM((_N,), jnp.float32),
        pltpu.VMEM((_CAP,), jnp.int32),
        pltpu.VMEM((_CAP,), jnp.int32),
        pltpu.VMEM((_K,), jnp.int32),
        pltpu.VMEM((_K,), jnp.int32),
        pltpu.SemaphoreType.DMA,
        pltpu.SemaphoreType.DMA,
        pltpu.SemaphoreType.DMA,
    ],
)


def _sc_topk_body(x_hbm, out_hbm, rowbuf_a, rowbuf_b, outbuf, candi, sbuf, ctmp, previdx,
             sem_a, sem_b, sem_o):
    wid = lax.axis_index("s") * _NC + lax.axis_index("c")
    r0 = wid * _RPW
    zf16 = jnp.zeros((16,), jnp.float32)

    def z_body(i, _):
        outbuf[pl.ds(i * 16, 16)] = zf16
        return 0

    lax.fori_loop(0, _CHUNKS, z_body, 0)

    sems = (sem_a, sem_b)
    bufs = (rowbuf_a, rowbuf_b)
    pltpu.make_async_copy(x_hbm.at[r0], rowbuf_a, sem_a).start()
    for j in range(_RPW):
        rj = r0 + j
        buf = bufs[j % 2]
        pltpu.make_async_copy(x_hbm.at[rj], buf, sems[j % 2]).wait()
        if j + 1 < _RPW:
            pltpu.make_async_copy(x_hbm.at[rj + 1], bufs[(j + 1) % 2],
                                  sems[(j + 1) % 2]).start()

        _scan_row(buf, candi, sbuf, ctmp)

        idxs, vals = [], []
        for i in range(_K // 16):
            iv = candi[pl.ds(i * 16, 16)]
            vv = plsc.load_gather(buf, [iv])
            idxs.append(iv)
            vals.append(jnp.maximum(vv, 0.0))

        if j > 0:
            pltpu.make_async_copy(outbuf, out_hbm.at[rj - 1], sem_o).wait()
            for i in range(_K // 16):
                pz = previdx[pl.ds(i * 16, 16)]
                plsc.store_scatter(outbuf, [pz], zf16)
        for i in range(_K // 16):
            plsc.store_scatter(outbuf, [idxs[i]], vals[i])
            previdx[pl.ds(i * 16, 16)] = idxs[i]
        pltpu.make_async_copy(outbuf, out_hbm.at[rj], sem_o).start()

    pltpu.make_async_copy(outbuf, out_hbm.at[r0 + _RPW - 1], sem_o).wait()


_sc_topk = pl.kernel(_sc_topk_body, **_KERNEL_KWARGS)


def kernel(x):
    return _sc_topk(x)


# speedup vs baseline: 8.7849x; 2.1622x over previous
import functools

import jax
import jax.numpy as jnp
from jax import lax
from jax.experimental import pallas as pl
from jax.experimental.pallas import tpu as pltpu
from jax.experimental.pallas import tpu_sc as plsc

_K = 64
_N = 32768
_ROWS = 128
_NC = 2
_NS = 16
_NW = _NC * _NS
_RPW = _ROWS // _NW
_CHUNKS = _N // 16
_CKPTS = 32
_CPC = _CHUNKS // _CKPTS
_CAP = _CKPTS * 16 + _CPC * 16 + 16
_TRIG = 256
_IMIN = -2147483648


def _sortable(v):
    k = lax.bitcast_convert_type(v, jnp.int32)
    return jnp.where(k >= 0, k, k ^ jnp.int32(0x7FFFFFFF))


def _unsortable(s):
    k = jnp.where(s >= 0, s, s ^ jnp.int32(0x7FFFFFFF))
    return lax.bitcast_convert_type(k, jnp.float32)


def _splat(x, dtype=jnp.int32):
    return lax.broadcast(jnp.asarray(x, dtype), (16,))


def _rebuild(buf, candi, sbuf, ctmp, n_splat):
    iota16 = lax.iota(jnp.int32, 16)
    zeros_i = jnp.zeros((16,), jnp.int32)
    n_s = jnp.max(n_splat)
    nv = (n_s + 15) // 16

    def g_body(vi, _):
        base = vi * 16
        idxv = candi[pl.ds(base, 16)]
        valid = (lax.broadcast(base, (16,)) + iota16) < n_splat
        vals = plsc.load_gather(buf, [idxv], mask=valid)
        s = jnp.where(valid, _sortable(vals), jnp.int32(_IMIN))
        sbuf[pl.ds(base, 16)] = s
        return 0

    lax.fori_loop(0, nv, g_body, 0)

    def bit_body(i, p_u):
        cand_u = p_u | _splat(jnp.int32(1) << (31 - i))
        cand_s = cand_u ^ jnp.int32(_IMIN)

        def cnt_body(vi, acc):
            s = sbuf[pl.ds(vi * 16, 16)]
            return acc + plsc.all_reduce_population_count(s >= cand_s)

        cnt = lax.fori_loop(0, nv, cnt_body, zeros_i)
        return jnp.where(cnt >= _K, cand_u, p_u)

    p_u = lax.fori_loop(0, 32, bit_body, zeros_i)
    v64s = p_u ^ jnp.int32(_IMIN)

    def cgt_body(vi, acc):
        s = sbuf[pl.ds(vi * 16, 16)]
        return acc + plsc.all_reduce_population_count(s > v64s)

    c_gt = lax.fori_loop(0, nv, cgt_body, zeros_i)
    m_allow = _splat(_K) - c_gt

    def cpt_body(vi, carry):
        eq_seen, nn = carry
        base = vi * 16
        s = sbuf[pl.ds(base, 16)]
        idxv = candi[pl.ds(base, 16)]
        gt = s > v64s
        eq = s == v64s
        cum_eq = jnp.cumsum(jnp.where(eq, 1, 0))
        keep = gt | (eq & ((cum_eq + eq_seen) <= m_allow))
        slot = nn + jnp.cumsum(jnp.where(keep, 1, 0)) - 1
        plsc.store_scatter(ctmp, [slot], idxv, mask=keep)
        return (eq_seen + plsc.all_reduce_population_count(eq),
                nn + plsc.all_reduce_population_count(keep))

    lax.fori_loop(0, nv, cpt_body, (zeros_i, zeros_i))

    for i in range(_K // 16):
        candi[pl.ds(i * 16, 16)] = ctmp[pl.ds(i * 16, 16)]
    return _splat(_K), _unsortable(v64s)


def _cheap_compact(buf, candi, n_splat, thresh):
    iota16 = lax.iota(jnp.int32, 16)
    neg_inf = jnp.float32(-jnp.inf)
    n_s = jnp.max(n_splat)
    nfull = n_s // 16

    def t_body(vi, t):
        idxv = candi[pl.ds(vi * 16, 16)]
        vals = plsc.load_gather(buf, [idxv])
        sv = jnp.sort(vals)
        r5 = jnp.max(jnp.where(iota16 == 11, sv, neg_inf))
        return jnp.minimum(t, r5)

    t = lax.fori_loop(0, nfull, t_body, jnp.float32(jnp.inf))
    t_splat = lax.broadcast(t, (16,))
    nv = (n_s + 15) // 16

    def c_body(vi, nn):
        base = vi * 16
        idxv = candi[pl.ds(base, 16)]
        valid = (lax.broadcast(base, (16,)) + iota16) < n_splat
        vals = plsc.load_gather(buf, [idxv], mask=valid)
        keep = jnp.where(valid, vals, neg_inf) >= t_splat
        slot = nn + jnp.cumsum(jnp.where(keep, 1, 0)) - 1
        plsc.store_scatter(candi, [slot], idxv, mask=keep)
        return nn + plsc.all_reduce_population_count(keep)

    nn = lax.fori_loop(0, nv, c_body, jnp.zeros((16,), jnp.int32))
    return nn, jnp.maximum(thresh, t_splat)


_UNROLL = 8


def _scan_row(buf, candi, sbuf, ctmp):
    iota16 = lax.iota(jnp.int32, 16)
    c16 = _splat(16)
    gpc = _CPC // _UNROLL

    def ckpt_body(kc, carry):
        n_splat, thresh, idxb = carry

        def group_body(g, c):
            n_sp, ib = c
            ms, cums, pcs, idxs = [], [], [], []
            for u in range(_UNROLL):
                v = buf[pl.ds(g * (16 * _UNROLL) + u * 16, 16)]
                m = v > thresh
                ms.append(m)
                cums.append(jnp.cumsum(jnp.where(m, 1, 0)))
                pcs.append(plsc.all_reduce_population_count(m))
                idxs.append(ib)
                ib = ib + c16
            offm1 = n_sp - 1
            for u in range(_UNROLL):
                plsc.store_scatter(candi, [offm1 + cums[u]], idxs[u],
                                   mask=ms[u])
                offm1 = offm1 + pcs[u]
            return offm1 + 1, ib

        n_splat, idxb = lax.fori_loop(kc * gpc, (kc + 1) * gpc, group_body,
                                      (n_splat, idxb))
        n_s = jnp.max(n_splat)

        def do_compact(c):
            n_sp, th = c
            nn, th2 = _cheap_compact(buf, candi, n_sp, th)
            return lax.cond(
                jnp.max(nn) >= _TRIG,
                lambda cc: _rebuild(buf, candi, sbuf, ctmp, cc[0]),
                lambda cc: cc,
                (nn, th2))

        n_splat, thresh = lax.cond(n_s >= _TRIG, do_compact, lambda c: c,
                                   (n_splat, thresh))
        return n_splat, thresh, idxb

    n0 = jnp.zeros((16,), jnp.int32)
    t0 = jnp.full((16,), -jnp.inf, jnp.float32)
    n_splat, _, _ = lax.fori_loop(0, _CKPTS, ckpt_body, (n0, t0, iota16))
    _rebuild(buf, candi, sbuf, ctmp, n_splat)


_mesh = plsc.VectorSubcoreMesh(core_axis_name="c", subcore_axis_name="s")


_KERNEL_KWARGS = dict(
    mesh=_mesh,
    compiler_params=pltpu.CompilerParams(needs_layout_passes=False),
    out_type=jax.ShapeDtypeStruct((_ROWS, _N), jnp.float32),
    scratch_types=[
        pltpu.VMEM((_N,), jnp.float32),
        pltpu.VMEM((_N,), jnp.float32),
        pltpu.VMEM((_N,), jnp.float32),
        pltpu.VMEM((_CAP,), jnp.int32),
        pltpu.VMEM((_CAP,), jnp.int32),
        pltpu.VMEM((_K,), jnp.int32),
        pltpu.VMEM((_K,), jnp.int32),
        pltpu.SemaphoreType.DMA,
        pltpu.SemaphoreType.DMA,
        pltpu.SemaphoreType.DMA,
    ],
)


def _sc_topk_body(x_hbm, out_hbm, rowbuf_a, rowbuf_b, outbuf, candi, sbuf, ctmp, previdx,
             sem_a, sem_b, sem_o):
    wid = lax.axis_index("s") * _NC + lax.axis_index("c")
    r0 = wid * _RPW
    zf16 = jnp.zeros((16,), jnp.float32)

    def z_body(i, _):
        for u in range(8):
            outbuf[pl.ds(i * 128 + u * 16, 16)] = zf16
        return 0

    lax.fori_loop(0, _CHUNKS // 8, z_body, 0)

    sems = (sem_a, sem_b)
    bufs = (rowbuf_a, rowbuf_b)
    pltpu.make_async_copy(x_hbm.at[r0], rowbuf_a, sem_a).start()
    for j in range(_RPW):
        rj = r0 + j
        buf = bufs[j % 2]
        pltpu.make_async_copy(x_hbm.at[rj], buf, sems[j % 2]).wait()
        if j + 1 < _RPW:
            pltpu.make_async_copy(x_hbm.at[rj + 1], bufs[(j + 1) % 2],
                                  sems[(j + 1) % 2]).start()

        _scan_row(buf, candi, sbuf, ctmp)

        idxs, vals = [], []
        for i in range(_K // 16):
            iv = candi[pl.ds(i * 16, 16)]
            vv = plsc.load_gather(buf, [iv])
            idxs.append(iv)
            vals.append(jnp.maximum(vv, 0.0))

        if j > 0:
            pltpu.make_async_copy(outbuf, out_hbm.at[rj - 1], sem_o).wait()
            for i in range(_K // 16):
                pz = previdx[pl.ds(i * 16, 16)]
                plsc.store_scatter(outbuf, [pz], zf16)
        for i in range(_K // 16):
            plsc.store_scatter(outbuf, [idxs[i]], vals[i])
            previdx[pl.ds(i * 16, 16)] = idxs[i]
        pltpu.make_async_copy(outbuf, out_hbm.at[rj], sem_o).start()

    pltpu.make_async_copy(outbuf, out_hbm.at[r0 + _RPW - 1], sem_o).wait()


_sc_topk = pl.kernel(_sc_topk_body, **_KERNEL_KWARGS)


def kernel(x):
    return _sc_topk(x)
